# baseline (speedup 1.0000x reference)
import jax
import jax.numpy as jnp
from jax import lax
from jax.experimental import pallas as pl
from jax.experimental.pallas import tpu as pltpu
from jax.experimental.pallas import tpu_sc as plsc

_MARGIN = 1.0
_K = 256
_EPS = 1e-12
_INT_MIN = -2147483648
_SENT = 0x7f800000
_BLK = 128
_B = 1024
_NBLK = _B // _BLK
_NPB = _NBLK * (_NBLK + 1) // 2
_TOT = _NPB * _BLK * _BLK
_NSENT = _NBLK * (_BLK * (_BLK + 1) // 2)
_NPAIRS = _B * (_B - 1) // 2
_NT = 16
_PER = _TOT // _NT
_NV = _PER // 16
_MAGIC = 0x5f3759df


def _tc_body(emb_ref, lab_col_ref, lab_row_ref, out_ref, full_ref):
    emb = emb_ref[...]
    B = emb.shape[0]

    g = lax.dot_general(emb, emb, (((1,), (1,)), ((), ())),
                        preferred_element_type=jnp.float32)

    row = lax.broadcasted_iota(jnp.int32, (B, B), 0)
    col = lax.broadcasted_iota(jnp.int32, (B, B), 1)
    gd = jnp.where(row == col, g, 0.0)
    norms_col = jnp.sum(gd, axis=1, keepdims=True)
    norms_row = jnp.sum(gd, axis=0, keepdims=True)

    sq = jnp.maximum(norms_col + norms_row - 2.0 * g, 0.0)
    labmatch = lab_col_ref[...] == lab_row_ref[...]

    bits = jnp.maximum(lax.bitcast_convert_type(sq, jnp.int32), 0)
    enc = jnp.where(labmatch, bits, bits + jnp.int32(_INT_MIN))
    full_ref[...] = jnp.where(row < col, enc, jnp.int32(_SENT))

    idx = 0
    for bi in range(_NBLK):
        for bj in range(bi, _NBLK):
            out_ref[idx * _BLK:(idx + 1) * _BLK, :] = (
                full_ref[bi * _BLK:(bi + 1) * _BLK,
                         bj * _BLK:(bj + 1) * _BLK])
            idx += 1


def _vsqrt(x):
    i = lax.bitcast_convert_type(x, jnp.int32)
    y = lax.bitcast_convert_type(
        jnp.int32(_MAGIC) - lax.shift_right_logical(i, 1), jnp.float32)
    for _ in range(3):
        y = y * (1.5 - 0.5 * x * y * y)
    d = x * y
    return 0.5 * (d + x / d)


def _sc_body(enc_hbm, out_hbm,
             enc_v, hist_p, hist_n, comb_p, comb_n, zeros_v, ridx_v,
             part_v, part_rb, out_v,
             sh_p0, sh_n0, sh_p1, sh_n1, sh_part):
    wid = lax.axis_index("s")
    lane = lax.iota(jnp.int32, 16)
    ones = jnp.ones((16,), jnp.int32)
    zvec = jnp.zeros((16,), jnp.int32)

    pltpu.sync_copy(enc_hbm.at[pl.ds(wid * _PER, _PER)], enc_v)

    ridx_v[...] = lane

    def zero2d(ref):
        def zbody(l, c):
            for grp in range(16):
                ref[l, pl.ds(16 * grp, 16)] = zvec
            return c
        lax.fori_loop(0, 16, zbody, jnp.int32(0))

    zero2d(zeros_v)

    def fold(ref):
        def fbody(l, acc):
            return tuple(acc[grp] + ref[l, pl.ds(16 * grp, 16)]
                         for grp in range(16))
        return lax.fori_loop(
            0, 16, fbody, tuple(jnp.zeros((16,), jnp.int32)
                                for _ in range(16)))

    def descend_hi(tg, k_rem):
        gs = [jnp.sum(tg[grp]) for grp in range(16)]
        nb = jnp.int32(0)
        cnt_gt = jnp.int32(0)
        sg = jnp.int32(0)
        for grp in range(15, -1, -1):
            s = lax.rev(jnp.cumsum(lax.rev(tg[grp], (0,))), (0,)) + sg
            nb = nb + jnp.sum(jnp.where(s >= k_rem, 1, 0))
            cnt_gt = cnt_gt + jnp.sum(jnp.where(s < k_rem, tg[grp], 0))
            sg = sg + gs[grp]
        return nb - 1, cnt_gt, gs

    def descend_lo(tg, k_rem):
        nb = jnp.int32(0)
        cnt_lt = jnp.int32(0)
        pg = jnp.int32(0)
        for grp in range(16):
            p = jnp.cumsum(tg[grp]) + pg
            nb = nb + jnp.sum(jnp.where(p < k_rem, 1, 0))
            cnt_lt = cnt_lt + jnp.sum(jnp.where(p < k_rem, tg[grp], 0))
            pg = pg + jnp.sum(tg[grp])
        return nb, cnt_lt

    pref_p = jnp.int32(0)
    pref_n = jnp.int32(0)
    k_p = jnp.int32(_K + _NSENT)
    k_n = jnp.int32(_K)
    n_pos_i = jnp.int32(0)

    shared_sets = [(sh_p0, sh_n0), (sh_p1, sh_n1)]

    for p in range(4):
        shift = 24 - 8 * p
        sh_p, sh_n = shared_sets[p % 2]

        zero2d(hist_p)
        if p > 0:
            zero2d(hist_n)

        @pl.when(wid == 0)
        def _():
            pltpu.sync_copy(zeros_v, sh_p)
            if p > 0:
                pltpu.sync_copy(zeros_v, sh_n)

        plsc.subcore_barrier()

        if p > 0:
            hi_p = lax.shift_right_logical(pref_p, shift + 8)
            hi_n = lax.shift_right_logical(pref_n, shift + 8)
            hi_p_v = lax.broadcast_in_dim(hi_p, (16,), ())
            hi_n_v = lax.broadcast_in_dim(hi_n, (16,), ())

        def scan_body(i, c):
            e = enc_v[pl.ds(i * 16, 16)]
            u = jnp.bitwise_xor(e, jnp.int32(_INT_MIN))
            dig = jnp.bitwise_and(
                lax.shift_right_logical(u, shift), jnp.int32(255))
            if p == 0:
                plsc.addupdate_scatter(hist_p, [lane, dig], ones)
            else:
                top = lax.shift_right_logical(u, shift + 8)
                plsc.addupdate_scatter(hist_p, [lane, dig], ones,
                                       mask=top == hi_p_v)
                plsc.addupdate_scatter(hist_n, [lane, dig], ones,
                                       mask=top == hi_n_v)
            return c

        lax.fori_loop(0, _NV, scan_body, jnp.int32(0))

        pltpu.sync_copy(hist_p, sh_p.at[ridx_v], add=True)
        if p > 0:
            pltpu.sync_copy(hist_n, sh_n.at[ridx_v], add=True)

        plsc.subcore_barrier()

        pltpu.sync_copy(sh_p, comb_p)
        tg_p = fold(comb_p)
        if p > 0:
            pltpu.sync_copy(sh_n, comb_n)
            tg_n = fold(comb_n)
        else:
            tg_n = tg_p

        b_p, cnt_gt, gs = descend_hi(tg_p, k_p)
        k_p = k_p - cnt_gt
        pref_p = jnp.bitwise_or(pref_p, lax.shift_left(b_p, shift))
        if p == 0:
            n_pos_i = sum(gs[8:]) - jnp.int32(_NSENT)

        b_n, cnt_lt = descend_lo(tg_n, k_n)
        k_n = k_n - cnt_lt
        pref_n = jnp.bitwise_or(pref_n, lax.shift_left(b_n, shift))

    t_e_p = jnp.bitwise_xor(pref_p, jnp.int32(_INT_MIN))
    t_e_n = jnp.bitwise_xor(pref_n, jnp.int32(_INT_MIN))
    t_e_p_v = lax.broadcast_in_dim(t_e_p, (16,), ())
    t_e_n_v = lax.broadcast_in_dim(t_e_n, (16,), ())
    sent_v = lax.broadcast_in_dim(jnp.int32(_SENT), (16,), ())
    tval_p = lax.bitcast_convert_type(t_e_p_v, jnp.float32)
    tval_n = lax.bitcast_convert_type(
        lax.broadcast_in_dim(pref_n, (16,), ()), jnp.float32)

    zf = jnp.zeros((16,), jnp.float32)
    onesf = jnp.ones((16,), jnp.float32)

    def sum_body(i, c):
        s_p, c_p, s_nl, c_nl, s_na = c
        e = enc_v[pl.ds(i * 16, 16)]
        vbits = jnp.where(e >= 0, e, e - jnp.int32(_INT_MIN))
        v = lax.bitcast_convert_type(vbits, jnp.float32)
        pgt = (e > t_e_p_v) & (e < sent_v)
        s_p = s_p + jnp.where(pgt, v, zf)
        c_p = c_p + jnp.where(pgt, onesf, zf)
        isneg = e < 0
        active = isneg & (v < jnp.float32(_MARGIN * _MARGIN))
        d = _vsqrt(v + jnp.float32(_EPS))
        r = jnp.maximum(jnp.float32(_MARGIN) - d, 0.0)
        fv = jnp.where(active, r * r, zf)
        nlt = e < t_e_n_v
        s_nl = s_nl + jnp.where(nlt, fv, zf)
        c_nl = c_nl + jnp.where(nlt, onesf, zf)
        s_na = s_na + fv
        return s_p, c_p, s_nl, c_nl, s_na

    s_p, c_p, s_nl, c_nl, s_na = lax.fori_loop(
        0, _NV, sum_body, (zf, zf, zf, zf, zf))

    def splat(x):
        return lax.broadcast_in_dim(jnp.sum(x), (16,), ())

    lane_f = lane
    part = (jnp.where(lane_f == 0, splat(s_p), zf)
            + jnp.where(lane_f == 1, splat(c_p), zf)
            + jnp.where(lane_f == 2, splat(s_nl), zf)
            + jnp.where(lane_f == 3, splat(c_nl), zf)
            + jnp.where(lane_f == 4, splat(s_na), zf))
    part_v[...] = part
    pltpu.sync_copy(part_v, sh_part.at[wid])
    plsc.subcore_barrier()

    pltpu.sync_copy(sh_part, part_rb)

    def pfold(t, acc):
        return acc + part_rb[t, pl.ds(0, 16)]
    tot = lax.fori_loop(0, 16, pfold, zf)

    def pick(i):
        return lax.broadcast_in_dim(
            jnp.sum(jnp.where(lane == i, tot, zf)), (16,), ())

    g_sp, g_cp, g_snl, g_cnl, g_sna = (pick(0), pick(1), pick(2),
                                       pick(3), pick(4))

    kf = jnp.float32(_K)
    n_pos_f = lax.broadcast_in_dim(n_pos_i, (16,), ()).astype(jnp.float32)
    n_neg_f = jnp.float32(_NPAIRS) - n_pos_f

    pos_sum = g_sp + (kf - g_cp) * tval_p + kf * jnp.float32(_EPS)

    dt = _vsqrt(tval_n + jnp.float32(_EPS))
    rt = jnp.maximum(jnp.float32(_MARGIN) - dt, 0.0)
    neg_topk = g_snl + (kf - g_cnl) * rt * rt
    neg_sum = jnp.where(n_neg_f > kf, neg_topk, g_sna)

    count = jnp.minimum(n_pos_f, kf) + jnp.minimum(n_neg_f, kf)
    out_v[...] = (pos_sum + neg_sum) / count

    @pl.when(wid == 0)
    def _():
        pltpu.sync_copy(out_v, out_hbm)


@jax.jit
def kernel(embeddings, labels):
    B = embeddings.shape[0]
    labels = labels.astype(jnp.int32)
    lab_col = labels.reshape(B, 1)
    lab_row = labels.reshape(1, B)

    enc = pl.pallas_call(
        _tc_body,
        out_shape=jax.ShapeDtypeStruct((_NPB * _BLK, _BLK), jnp.int32),
        scratch_shapes=[pltpu.VMEM((B, B), jnp.int32)],
    )(embeddings, lab_col, lab_row)

    mesh = plsc.VectorSubcoreMesh(core_axis_name="c", subcore_axis_name="s",
                                  num_cores=1, num_subcores=_NT)
    sc = pl.kernel(
        _sc_body,
        out_type=jax.ShapeDtypeStruct((16,), jnp.float32),
        mesh=mesh,
        compiler_params=pltpu.CompilerParams(use_tc_tiling_on_sc=False,
                                             needs_layout_passes=False),
        scratch_types=[
            pltpu.VMEM((_PER,), jnp.int32),
            pltpu.VMEM((16, 256), jnp.int32),
            pltpu.VMEM((16, 256), jnp.int32),
            pltpu.VMEM((16, 256), jnp.int32),
            pltpu.VMEM((16, 256), jnp.int32),
            pltpu.VMEM((16, 256), jnp.int32),
            pltpu.VMEM((16,), jnp.int32),
            pltpu.VMEM((16,), jnp.float32),
            pltpu.VMEM((16, 16), jnp.float32),
            pltpu.VMEM((16,), jnp.float32),
            pltpu.VMEM_SHARED((16, 256), jnp.int32),
            pltpu.VMEM_SHARED((16, 256), jnp.int32),
            pltpu.VMEM_SHARED((16, 256), jnp.int32),
            pltpu.VMEM_SHARED((16, 256), jnp.int32),
            pltpu.VMEM_SHARED((16, 16), jnp.float32),
        ],
    )
    loss16 = sc(enc.reshape(_TOT))
    return loss16[0]
